# baseline (speedup 1.0000x reference)
import jax
import jax.numpy as jnp
from jax import lax
from jax.experimental import pallas as pl

N = 10000
E = 320000
D = 128
ED = 16
H = 128
G = 64


def _edge_mlp_body(ea_ref, lw0_ref, lb0_ref, lw1_ref, lb1_ref, e0_ref, e1_ref):
    ea = ea_ref[...]
    e0_ref[...] = jnp.dot(ea, lw0_ref[...], preferred_element_type=jnp.float32) + lb0_ref[...]
    e1_ref[...] = jnp.dot(ea, lw1_ref[...], preferred_element_type=jnp.float32) + lb1_ref[...]


def _edge_mlp(edge_attr, lew0, leb0, lew1, leb1):
    TE = 4000
    return pl.pallas_call(
        _edge_mlp_body,
        grid=(E // TE,),
        in_specs=[
            pl.BlockSpec((TE, ED), lambda i: (i, 0)),
            pl.BlockSpec((ED, D), lambda i: (0, 0)),
            pl.BlockSpec((1, D), lambda i: (0, 0)),
            pl.BlockSpec((ED, D), lambda i: (0, 0)),
            pl.BlockSpec((1, D), lambda i: (0, 0)),
        ],
        out_specs=[
            pl.BlockSpec((TE, D), lambda i: (i, 0)),
            pl.BlockSpec((TE, D), lambda i: (i, 0)),
        ],
        out_shape=[
            jax.ShapeDtypeStruct((E, D), jnp.float32),
            jax.ShapeDtypeStruct((E, D), jnp.float32),
        ],
    )(edge_attr, lew0, leb0.reshape(1, D), lew1, leb1.reshape(1, D))


def _mlp_body(x_ref, a_ref, w1_ref, b1_ref, w2_ref, b2_ref,
              z_ref, s1_ref, s2_ref):
    h = x_ref[...] + a_ref[...]
    t = jnp.maximum(jnp.dot(h, w1_ref[...], preferred_element_type=jnp.float32) + b1_ref[...], 0.0)
    z = jnp.dot(t, w2_ref[...], preferred_element_type=jnp.float32) + b2_ref[...]
    z = jnp.maximum(z, 0.0)
    z_ref[...] = z

    @pl.when(pl.program_id(0) == 0)
    def _():
        s1_ref[...] = jnp.zeros_like(s1_ref)
        s2_ref[...] = jnp.zeros_like(s2_ref)

    s1_ref[...] += jnp.sum(z, axis=0, keepdims=True)
    s2_ref[...] += jnp.sum(z * z, axis=0, keepdims=True)


def _node_mlp(x, agg, w1, b1, w2, b2):
    TN = 1000
    return pl.pallas_call(
        _mlp_body,
        grid=(N // TN,),
        in_specs=[
            pl.BlockSpec((TN, D), lambda i: (i, 0)),
            pl.BlockSpec((TN, D), lambda i: (i, 0)),
            pl.BlockSpec((D, H), lambda i: (0, 0)),
            pl.BlockSpec((1, H), lambda i: (0, 0)),
            pl.BlockSpec((H, H), lambda i: (0, 0)),
            pl.BlockSpec((1, H), lambda i: (0, 0)),
        ],
        out_specs=[
            pl.BlockSpec((TN, H), lambda i: (i, 0)),
            pl.BlockSpec((1, H), lambda i: (0, 0)),
            pl.BlockSpec((1, H), lambda i: (0, 0)),
        ],
        out_shape=[
            jax.ShapeDtypeStruct((N, H), jnp.float32),
            jax.ShapeDtypeStruct((1, H), jnp.float32),
            jax.ShapeDtypeStruct((1, H), jnp.float32),
        ],
    )(x, agg, w1, b1.reshape(1, H), w2, b2.reshape(1, H))


def _bn_pool_body(z_ref, s1_ref, s2_ref, g_ref, b_ref, batch_ref,
                  zb_ref, p_ref):
    mu = s1_ref[...] / N
    var = s2_ref[...] / N - mu * mu
    scale = g_ref[...] * lax.rsqrt(var + 1e-5)
    shift = b_ref[...] - mu * scale
    zb = z_ref[...] * scale + shift
    zb_ref[...] = zb

    bt = batch_ref[0]
    oh = (lax.broadcasted_iota(jnp.int32, (G, bt.shape[1]), 0) == bt).astype(jnp.float32)
    part = jnp.dot(oh, zb, preferred_element_type=jnp.float32)

    @pl.when(pl.program_id(0) == 0)
    def _():
        p_ref[...] = jnp.zeros_like(p_ref)

    p_ref[...] += part


def _bn_pool(z, s1, s2, gamma, beta, batch3):
    TN = 1000
    return pl.pallas_call(
        _bn_pool_body,
        grid=(N // TN,),
        in_specs=[
            pl.BlockSpec((TN, H), lambda i: (i, 0)),
            pl.BlockSpec((1, H), lambda i: (0, 0)),
            pl.BlockSpec((1, H), lambda i: (0, 0)),
            pl.BlockSpec((1, H), lambda i: (0, 0)),
            pl.BlockSpec((1, H), lambda i: (0, 0)),
            pl.BlockSpec((1, 1, TN), lambda i: (i, 0, 0)),
        ],
        out_specs=[
            pl.BlockSpec((TN, H), lambda i: (i, 0)),
            pl.BlockSpec((G, H), lambda i: (0, 0)),
        ],
        out_shape=[
            jax.ShapeDtypeStruct((N, H), jnp.float32),
            jax.ShapeDtypeStruct((G, H), jnp.float32),
        ],
    )(z, s1, s2, gamma.reshape(1, H), beta.reshape(1, H), batch3)


def _agg(x, src, dst, e):
    m = jax.nn.relu(x[src] + e)
    return jax.ops.segment_sum(m, dst, num_segments=N)


def kernel(x, edge_index, edge_attr, batch, lew0, leb0, w1_0, b1_0, w2_0, b2_0,
           bng0, bnb0, lew1, leb1, w1_1, b1_1, w2_1, b2_1, bng1, bnb1):
    src = edge_index[0]
    dst = edge_index[1]
    batch3 = batch.reshape(N // 1000, 1, 1000)

    e0, e1 = _edge_mlp(edge_attr, lew0, leb0, lew1, leb1)

    z0, s1, s2 = _node_mlp(x, _agg(x, src, dst, e0), w1_0, b1_0, w2_0, b2_0)
    zb0, g0 = _bn_pool(z0, s1, s2, bng0, bnb0, batch3)

    z1, s1, s2 = _node_mlp(zb0, _agg(zb0, src, dst, e1), w1_1, b1_1, w2_1, b2_1)
    zb1, g1 = _bn_pool(z1, s1, s2, bng1, bnb1, batch3)

    zcat = jnp.concatenate([zb0, zb1], axis=1)
    gcat = jnp.concatenate([g0, g1], axis=1)
    return (zcat, gcat)


# speedup vs baseline: 1.0420x; 1.0420x over previous
import jax
import jax.numpy as jnp
from jax import lax
from jax.experimental import pallas as pl

N = 10000
E = 320000
D = 128
ED = 16
H = 128
G = 64


def _edge_mlp_body(ea_ref, lw0_ref, lb0_ref, lw1_ref, lb1_ref, e0_ref, e1_ref):
    ea = ea_ref[...]
    e0_ref[...] = jnp.dot(ea, lw0_ref[...], preferred_element_type=jnp.float32) + lb0_ref[...]
    e1_ref[...] = jnp.dot(ea, lw1_ref[...], preferred_element_type=jnp.float32) + lb1_ref[...]


def _edge_mlp(edge_attr, lew0, leb0, lew1, leb1):
    TE = 4000
    return pl.pallas_call(
        _edge_mlp_body,
        grid=(E // TE,),
        in_specs=[
            pl.BlockSpec((TE, ED), lambda i: (i, 0)),
            pl.BlockSpec((ED, D), lambda i: (0, 0)),
            pl.BlockSpec((1, D), lambda i: (0, 0)),
            pl.BlockSpec((ED, D), lambda i: (0, 0)),
            pl.BlockSpec((1, D), lambda i: (0, 0)),
        ],
        out_specs=[
            pl.BlockSpec((TE, D), lambda i: (i, 0)),
            pl.BlockSpec((TE, D), lambda i: (i, 0)),
        ],
        out_shape=[
            jax.ShapeDtypeStruct((E, D), jnp.float32),
            jax.ShapeDtypeStruct((E, D), jnp.float32),
        ],
    )(edge_attr, lew0, leb0.reshape(1, D), lew1, leb1.reshape(1, D))


def _mlp_body(x_ref, a_ref, w1_ref, b1_ref, w2_ref, b2_ref,
              z_ref, s1_ref, s2_ref):
    h = x_ref[...] + a_ref[...]
    t = jnp.maximum(jnp.dot(h, w1_ref[...], preferred_element_type=jnp.float32) + b1_ref[...], 0.0)
    z = jnp.dot(t, w2_ref[...], preferred_element_type=jnp.float32) + b2_ref[...]
    z = jnp.maximum(z, 0.0)
    z_ref[...] = z

    @pl.when(pl.program_id(0) == 0)
    def _():
        s1_ref[...] = jnp.zeros_like(s1_ref)
        s2_ref[...] = jnp.zeros_like(s2_ref)

    s1_ref[...] += jnp.sum(z, axis=0, keepdims=True)
    s2_ref[...] += jnp.sum(z * z, axis=0, keepdims=True)


def _node_mlp(x, agg, w1, b1, w2, b2):
    TN = 1000
    return pl.pallas_call(
        _mlp_body,
        grid=(N // TN,),
        in_specs=[
            pl.BlockSpec((TN, D), lambda i: (i, 0)),
            pl.BlockSpec((TN, D), lambda i: (i, 0)),
            pl.BlockSpec((D, H), lambda i: (0, 0)),
            pl.BlockSpec((1, H), lambda i: (0, 0)),
            pl.BlockSpec((H, H), lambda i: (0, 0)),
            pl.BlockSpec((1, H), lambda i: (0, 0)),
        ],
        out_specs=[
            pl.BlockSpec((TN, H), lambda i: (i, 0)),
            pl.BlockSpec((1, H), lambda i: (0, 0)),
            pl.BlockSpec((1, H), lambda i: (0, 0)),
        ],
        out_shape=[
            jax.ShapeDtypeStruct((N, H), jnp.float32),
            jax.ShapeDtypeStruct((1, H), jnp.float32),
            jax.ShapeDtypeStruct((1, H), jnp.float32),
        ],
    )(x, agg, w1, b1.reshape(1, H), w2, b2.reshape(1, H))


def _bn_pool_body(z_ref, s1_ref, s2_ref, g_ref, b_ref, batch_ref,
                  zb_ref, p_ref):
    mu = s1_ref[...] / N
    var = s2_ref[...] / N - mu * mu
    scale = g_ref[...] * lax.rsqrt(var + 1e-5)
    shift = b_ref[...] - mu * scale
    zb = z_ref[...] * scale + shift
    zb_ref[...] = zb

    bt = batch_ref[0]
    oh = (lax.broadcasted_iota(jnp.int32, (G, bt.shape[1]), 0) == bt).astype(jnp.float32)
    part = jnp.dot(oh, zb, preferred_element_type=jnp.float32)

    @pl.when(pl.program_id(0) == 0)
    def _():
        p_ref[...] = jnp.zeros_like(p_ref)

    p_ref[...] += part


def _bn_pool(z, s1, s2, gamma, beta, batch3):
    TN = 1000
    return pl.pallas_call(
        _bn_pool_body,
        grid=(N // TN,),
        in_specs=[
            pl.BlockSpec((TN, H), lambda i: (i, 0)),
            pl.BlockSpec((1, H), lambda i: (0, 0)),
            pl.BlockSpec((1, H), lambda i: (0, 0)),
            pl.BlockSpec((1, H), lambda i: (0, 0)),
            pl.BlockSpec((1, H), lambda i: (0, 0)),
            pl.BlockSpec((1, 1, TN), lambda i: (i, 0, 0)),
        ],
        out_specs=[
            pl.BlockSpec((TN, H), lambda i: (i, 0)),
            pl.BlockSpec((G, H), lambda i: (0, 0)),
        ],
        out_shape=[
            jax.ShapeDtypeStruct((N, H), jnp.float32),
            jax.ShapeDtypeStruct((G, H), jnp.float32),
        ],
    )(z, s1, s2, gamma.reshape(1, H), beta.reshape(1, H), batch3)


def _msg_body(xg_ref, ea_ref, lw_ref, lb_ref, m_ref):
    e = jnp.dot(ea_ref[...], lw_ref[...], preferred_element_type=jnp.float32) + lb_ref[...]
    m_ref[...] = jnp.maximum(xg_ref[...] + e, 0.0)


def _msg(xg, ea, lw, lb):
    TE = 4000
    return pl.pallas_call(
        _msg_body,
        grid=(E // TE,),
        in_specs=[
            pl.BlockSpec((TE, D), lambda i: (i, 0)),
            pl.BlockSpec((TE, ED), lambda i: (i, 0)),
            pl.BlockSpec((ED, D), lambda i: (0, 0)),
            pl.BlockSpec((1, D), lambda i: (0, 0)),
        ],
        out_specs=pl.BlockSpec((TE, D), lambda i: (i, 0)),
        out_shape=jax.ShapeDtypeStruct((E, D), jnp.float32),
    )(xg, ea, lw, lb.reshape(1, D))


def _agg(x, src, dst, ea, lw, lb):
    m = _msg(x[src], ea, lw, lb)
    return jax.ops.segment_sum(m, dst, num_segments=N)


def kernel(x, edge_index, edge_attr, batch, lew0, leb0, w1_0, b1_0, w2_0, b2_0,
           bng0, bnb0, lew1, leb1, w1_1, b1_1, w2_1, b2_1, bng1, bnb1):
    src = edge_index[0]
    dst = edge_index[1]
    batch3 = batch.reshape(N // 1000, 1, 1000)

    z0, s1, s2 = _node_mlp(x, _agg(x, src, dst, edge_attr, lew0, leb0), w1_0, b1_0, w2_0, b2_0)
    zb0, g0 = _bn_pool(z0, s1, s2, bng0, bnb0, batch3)

    z1, s1, s2 = _node_mlp(zb0, _agg(zb0, src, dst, edge_attr, lew1, leb1), w1_1, b1_1, w2_1, b2_1)
    zb1, g1 = _bn_pool(z1, s1, s2, bng1, bnb1, batch3)

    zcat = jnp.concatenate([zb0, zb1], axis=1)
    gcat = jnp.concatenate([g0, g1], axis=1)
    return (zcat, gcat)
